# baseline (speedup 1.0000x reference)
import functools

import jax
import jax.numpy as jnp
from jax import lax
from jax.experimental import pallas as pl
from jax.experimental.pallas import tpu as pltpu
from jax.experimental.pallas import tpu_sc as plsc

_N = 10000
_D = 128
_EPS = 1e-7
_NC = 2
_NS = 16
_CHUNK = 32
_P = 10112
_PA = 10008
_RPT = 624


_NBUF = 10


def _edge_body(G):
    def body(tbl, srcs, dsts, zeros, out, sidx, didx, *rest):
        rows = rest[:_NBUF]
        acc = rest[_NBUF]
        gsem = rest[_NBUF + 1:_NBUF + 1 + _NBUF]
        ssem = rest[_NBUF + 1 + _NBUF:_NBUF + 1 + 2 * _NBUF]
        isem = rest[_NBUF + 1 + 2 * _NBUF]
        c = lax.axis_index("c")
        s = lax.axis_index("s")

        @pl.when(s < _NS - 1)
        def _():
            pltpu.sync_copy(zeros.at[pl.ds(0, _RPT)],
                            acc.at[pl.ds(s * _RPT, _RPT)])

        @pl.when(s == _NS - 1)
        def _():
            pltpu.sync_copy(zeros, acc.at[pl.ds((_NS - 1) * _RPT, 640)])

        pltpu.async_copy(srcs.at[c, s, 0], sidx.at[0], isem)
        pltpu.async_copy(dsts.at[s, 0], didx.at[0], isem)
        plsc.subcore_barrier()

        def step(g, carry):
            bank = g % 2
            pltpu.make_async_copy(srcs.at[c, s, g], sidx.at[bank], isem).wait()
            pltpu.make_async_copy(dsts.at[s, g], didx.at[bank], isem).wait()
            for b in range(_NBUF):
                @pl.when(g > 0)
                def _(b=b):
                    pltpu.make_async_copy(
                        rows[b], acc.at[didx.at[1 - bank, b]], ssem[b]).wait()
                pltpu.async_copy(tbl.at[sidx.at[bank, b]], rows[b], gsem[b])
            pltpu.async_copy(srcs.at[c, s, g + 1], sidx.at[1 - bank], isem)
            pltpu.async_copy(dsts.at[s, g + 1], didx.at[1 - bank], isem)
            for b in range(_NBUF):
                pltpu.make_async_copy(tbl.at[sidx.at[bank, b]], rows[b],
                                      gsem[b]).wait()
                pltpu.async_copy(rows[b], acc.at[didx.at[bank, b]],
                                 ssem[b], add=True)
            return carry

        lax.fori_loop(0, G, step, 0)
        pltpu.make_async_copy(srcs.at[c, s, G], sidx.at[G % 2], isem).wait()
        pltpu.make_async_copy(dsts.at[s, G], didx.at[G % 2], isem).wait()
        for b in range(_NBUF):
            pltpu.make_async_copy(rows[b], acc.at[didx.at[(G - 1) % 2, b]],
                                  ssem[b]).wait()
        plsc.subcore_barrier()

        @pl.when(s < _NS - 1)
        def _():
            pltpu.sync_copy(acc.at[pl.ds(s * _RPT, _RPT)],
                            out.at[c, pl.ds(s * _RPT, _RPT)])

        @pl.when(s == _NS - 1)
        def _():
            pltpu.sync_copy(acc.at[pl.ds((_NS - 1) * _RPT, 640)],
                            out.at[c, pl.ds((_NS - 1) * _RPT, 640)])

    return body


@functools.lru_cache(maxsize=None)
def _make_edge_fn(G):
    mesh = plsc.VectorSubcoreMesh(core_axis_name="c", subcore_axis_name="s")
    return pl.kernel(
        _edge_body(G),
        out_type=jax.ShapeDtypeStruct((_NC, _P, _D), jnp.float32),
        mesh=mesh,
        scratch_types=(
            [pltpu.VMEM((2, _NBUF, _CHUNK), jnp.int32),
             pltpu.VMEM((2, _NBUF, _CHUNK), jnp.int32)]
            + [pltpu.VMEM((_CHUNK, _D), jnp.float32) for _ in range(_NBUF)]
            + [pltpu.VMEM_SHARED((_PA, _D), jnp.float32)]
            + [pltpu.SemaphoreType.DMA for _ in range(2 * _NBUF)]
            + [pltpu.SemaphoreType.DMA]
        ),
    )


def _write_tables(z, tbl_ref):
    u = z + _EPS
    w = jnp.exp(u)
    tbl_ref[pl.ds(0, _N), :] = u * w
    tbl_ref[pl.ds(_P, _N), :] = w


def _prep_body(x_ref, tbl_ref):
    _write_tables(jnp.maximum(x_ref[...], 0.0), tbl_ref)


@functools.lru_cache(maxsize=None)
def _make_prep():
    return pl.pallas_call(
        _prep_body,
        out_shape=jax.ShapeDtypeStruct((2 * _P, _D), jnp.float32),
    )


def _dense_body(make_tbl, x_ref, S_ref, W1_ref, b1_ref, g_ref, be_ref,
                W2_ref, b2_ref, L_ref, lb_ref, z_ref, *maybe_tbl):
    s1 = S_ref[0, pl.ds(0, _N), :]
    s2 = S_ref[1, pl.ds(0, _N), :]
    out = s1 / (s2 + 1e-16) + x_ref[...]
    h = jnp.dot(out, W1_ref[...], preferred_element_type=jnp.float32) + b1_ref[...]
    mu = jnp.mean(h, axis=0, keepdims=True)
    var = jnp.mean((h - mu) ** 2, axis=0, keepdims=True)
    h = (h - mu) * (g_ref[...] * jax.lax.rsqrt(var + 1e-5)) + be_ref[...]
    h = jnp.maximum(h, 0.0)
    y = jnp.dot(h, W2_ref[...], preferred_element_type=jnp.float32) + b2_ref[...]
    z = jnp.maximum(
        jnp.dot(y, L_ref[...], preferred_element_type=jnp.float32) + lb_ref[...],
        0.0)
    z_ref[...] = z
    if make_tbl:
        _write_tables(z, maybe_tbl[0])


@functools.lru_cache(maxsize=None)
def _make_dense(make_tbl):
    out_shape = [jax.ShapeDtypeStruct((_N, _D), jnp.float32)]
    if make_tbl:
        out_shape.append(jax.ShapeDtypeStruct((2 * _P, _D), jnp.float32))
    return pl.pallas_call(
        functools.partial(_dense_body, make_tbl),
        out_shape=out_shape,
    )


def kernel(x, edge_index,
           W1a, b1a, g1, be1, W1b, b1b, L1, lb1,
           W2a, b2a, g2, be2, W2b, b2b, L2, lb2,
           W3a, b3a, g3, be3, W3b, b3b, L3, lb3):
    E = edge_index.shape[1]
    per_tile = -(-E // _NS)
    G = -(-per_tile // (_CHUNK * _NBUF))
    J = G * _NBUF
    EP = _NS * J * _CHUNK
    pad = EP - E

    ei = edge_index.astype(jnp.int32)
    srcp = jnp.concatenate([ei[0], jnp.full((pad,), _N, jnp.int32)])
    dstp = jnp.concatenate([ei[1], jnp.full((pad,), _N, jnp.int32)])
    xtra = jnp.zeros((_NS, 1, _NBUF, _CHUNK), jnp.int32)
    srcs = jnp.stack([srcp, srcp + _P]).reshape(_NC, _NS, G, _NBUF, _CHUNK)
    srcs = jnp.concatenate(
        [srcs, jnp.broadcast_to(xtra[None], (_NC, _NS, 1, _NBUF, _CHUNK))],
        axis=2)
    dsts = dstp.reshape(_NS, G, _NBUF, _CHUNK)
    dsts = jnp.concatenate([dsts, xtra], axis=1)
    zeros = jnp.zeros((640, _D), jnp.float32)

    edge_fn = _make_edge_fn(G)
    dense_t = _make_dense(True)
    dense_f = _make_dense(False)

    def vec(v):
        return v.reshape(1, -1)

    tbl = _make_prep()(x)
    S = edge_fn(tbl, srcs, dsts, zeros)
    z, tbl = dense_t(x, S, W1a, vec(b1a), vec(g1), vec(be1), W1b, vec(b1b),
                     L1, vec(lb1))
    S = edge_fn(tbl, srcs, dsts, zeros)
    z, tbl = dense_t(z, S, W2a, vec(b2a), vec(g2), vec(be2), W2b, vec(b2b),
                     L2, vec(lb2))
    S = edge_fn(tbl, srcs, dsts, zeros)
    z = dense_f(z, S, W3a, vec(b3a), vec(g3), vec(be3), W3b, vec(b3b),
                L3, vec(lb3))[0]
    return z


# speedup vs baseline: 1.0628x; 1.0628x over previous
import functools

import jax
import jax.numpy as jnp
from jax import lax
from jax.experimental import pallas as pl
from jax.experimental.pallas import tpu as pltpu
from jax.experimental.pallas import tpu_sc as plsc

_N = 10000
_D = 128
_EPS = 1e-7
_NC = 2
_NS = 16
_CHUNK = 64
_P = 10112
_PA = 10008
_RPT = 624


_NBUF = 5


def _edge_body(G):
    def body(tbl, srcs, dsts, zeros, out, sidx, didx, *rest):
        rows = rest[:_NBUF]
        acc = rest[_NBUF]
        gsem = rest[_NBUF + 1:_NBUF + 1 + _NBUF]
        ssem = rest[_NBUF + 1 + _NBUF:_NBUF + 1 + 2 * _NBUF]
        isem = rest[_NBUF + 1 + 2 * _NBUF]
        c = lax.axis_index("c")
        s = lax.axis_index("s")

        @pl.when(s < _NS - 1)
        def _():
            pltpu.sync_copy(zeros.at[pl.ds(0, _RPT)],
                            acc.at[pl.ds(s * _RPT, _RPT)])

        @pl.when(s == _NS - 1)
        def _():
            pltpu.sync_copy(zeros, acc.at[pl.ds((_NS - 1) * _RPT, 640)])

        pltpu.async_copy(srcs.at[c, s, 0], sidx.at[0], isem)
        pltpu.async_copy(dsts.at[s, 0], didx.at[0], isem)
        plsc.subcore_barrier()

        def step(g, carry):
            bank = g % 2
            pltpu.make_async_copy(srcs.at[c, s, g], sidx.at[bank], isem).wait()
            pltpu.make_async_copy(dsts.at[s, g], didx.at[bank], isem).wait()
            for b in range(_NBUF):
                @pl.when(g > 0)
                def _(b=b):
                    pltpu.make_async_copy(
                        rows[b], acc.at[didx.at[1 - bank, b]], ssem[b]).wait()
                pltpu.async_copy(tbl.at[sidx.at[bank, b]], rows[b], gsem[b])
            pltpu.async_copy(srcs.at[c, s, g + 1], sidx.at[1 - bank], isem)
            pltpu.async_copy(dsts.at[s, g + 1], didx.at[1 - bank], isem)
            for b in range(_NBUF):
                pltpu.make_async_copy(tbl.at[sidx.at[bank, b]], rows[b],
                                      gsem[b]).wait()
                pltpu.async_copy(rows[b], acc.at[didx.at[bank, b]],
                                 ssem[b], add=True)
            return carry

        lax.fori_loop(0, G, step, 0)
        pltpu.make_async_copy(srcs.at[c, s, G], sidx.at[G % 2], isem).wait()
        pltpu.make_async_copy(dsts.at[s, G], didx.at[G % 2], isem).wait()
        for b in range(_NBUF):
            pltpu.make_async_copy(rows[b], acc.at[didx.at[(G - 1) % 2, b]],
                                  ssem[b]).wait()
        plsc.subcore_barrier()

        @pl.when(s < _NS - 1)
        def _():
            pltpu.sync_copy(acc.at[pl.ds(s * _RPT, _RPT)],
                            out.at[c, pl.ds(s * _RPT, _RPT)])

        @pl.when(s == _NS - 1)
        def _():
            pltpu.sync_copy(acc.at[pl.ds((_NS - 1) * _RPT, 640)],
                            out.at[c, pl.ds((_NS - 1) * _RPT, 640)])

    return body


@functools.lru_cache(maxsize=None)
def _make_edge_fn(G):
    mesh = plsc.VectorSubcoreMesh(core_axis_name="c", subcore_axis_name="s")
    return pl.kernel(
        _edge_body(G),
        out_type=jax.ShapeDtypeStruct((_NC, _P, _D), jnp.float32),
        mesh=mesh,
        scratch_types=(
            [pltpu.VMEM((2, _NBUF, _CHUNK), jnp.int32),
             pltpu.VMEM((2, _NBUF, _CHUNK), jnp.int32)]
            + [pltpu.VMEM((_CHUNK, _D), jnp.float32) for _ in range(_NBUF)]
            + [pltpu.VMEM_SHARED((_PA, _D), jnp.float32)]
            + [pltpu.SemaphoreType.DMA for _ in range(2 * _NBUF)]
            + [pltpu.SemaphoreType.DMA]
        ),
    )


def _write_tables(z, tbl_ref):
    u = z + _EPS
    w = jnp.exp(u)
    tbl_ref[pl.ds(0, _N), :] = u * w
    tbl_ref[pl.ds(_P, _N), :] = w


def _prep_body(x_ref, tbl_ref):
    _write_tables(jnp.maximum(x_ref[...], 0.0), tbl_ref)


@functools.lru_cache(maxsize=None)
def _make_prep():
    return pl.pallas_call(
        _prep_body,
        out_shape=jax.ShapeDtypeStruct((2 * _P, _D), jnp.float32),
    )


def _dense_body(make_tbl, x_ref, S_ref, W1_ref, b1_ref, g_ref, be_ref,
                W2_ref, b2_ref, L_ref, lb_ref, z_ref, *maybe_tbl):
    s1 = S_ref[0, pl.ds(0, _N), :]
    s2 = S_ref[1, pl.ds(0, _N), :]
    out = s1 / (s2 + 1e-16) + x_ref[...]
    h = jnp.dot(out, W1_ref[...], preferred_element_type=jnp.float32) + b1_ref[...]
    mu = jnp.mean(h, axis=0, keepdims=True)
    var = jnp.mean((h - mu) ** 2, axis=0, keepdims=True)
    h = (h - mu) * (g_ref[...] * jax.lax.rsqrt(var + 1e-5)) + be_ref[...]
    h = jnp.maximum(h, 0.0)
    y = jnp.dot(h, W2_ref[...], preferred_element_type=jnp.float32) + b2_ref[...]
    z = jnp.maximum(
        jnp.dot(y, L_ref[...], preferred_element_type=jnp.float32) + lb_ref[...],
        0.0)
    z_ref[...] = z
    if make_tbl:
        _write_tables(z, maybe_tbl[0])


@functools.lru_cache(maxsize=None)
def _make_dense(make_tbl):
    out_shape = [jax.ShapeDtypeStruct((_N, _D), jnp.float32)]
    if make_tbl:
        out_shape.append(jax.ShapeDtypeStruct((2 * _P, _D), jnp.float32))
    return pl.pallas_call(
        functools.partial(_dense_body, make_tbl),
        out_shape=out_shape,
    )


def kernel(x, edge_index,
           W1a, b1a, g1, be1, W1b, b1b, L1, lb1,
           W2a, b2a, g2, be2, W2b, b2b, L2, lb2,
           W3a, b3a, g3, be3, W3b, b3b, L3, lb3):
    E = edge_index.shape[1]
    per_tile = -(-E // _NS)
    G = -(-per_tile // (_CHUNK * _NBUF))
    J = G * _NBUF
    EP = _NS * J * _CHUNK
    pad = EP - E

    ei = edge_index.astype(jnp.int32)
    srcp = jnp.concatenate([ei[0], jnp.full((pad,), _N, jnp.int32)])
    dstp = jnp.concatenate([ei[1], jnp.full((pad,), _N, jnp.int32)])
    xtra = jnp.zeros((_NS, 1, _NBUF, _CHUNK), jnp.int32)
    srcs = jnp.stack([srcp, srcp + _P]).reshape(_NC, _NS, G, _NBUF, _CHUNK)
    srcs = jnp.concatenate(
        [srcs, jnp.broadcast_to(xtra[None], (_NC, _NS, 1, _NBUF, _CHUNK))],
        axis=2)
    dsts = dstp.reshape(_NS, G, _NBUF, _CHUNK)
    dsts = jnp.concatenate([dsts, xtra], axis=1)
    zeros = jnp.zeros((640, _D), jnp.float32)

    edge_fn = _make_edge_fn(G)
    dense_t = _make_dense(True)
    dense_f = _make_dense(False)

    def vec(v):
        return v.reshape(1, -1)

    tbl = _make_prep()(x)
    S = edge_fn(tbl, srcs, dsts, zeros)
    z, tbl = dense_t(x, S, W1a, vec(b1a), vec(g1), vec(be1), W1b, vec(b1b),
                     L1, vec(lb1))
    S = edge_fn(tbl, srcs, dsts, zeros)
    z, tbl = dense_t(z, S, W2a, vec(b2a), vec(g2), vec(be2), W2b, vec(b2b),
                     L2, vec(lb2))
    S = edge_fn(tbl, srcs, dsts, zeros)
    z = dense_f(z, S, W3a, vec(b3a), vec(g3), vec(be3), W3b, vec(b3b),
                L3, vec(lb3))[0]
    return z
